# Initial kernel scaffold; baseline (speedup 1.0000x reference)
import jax
import jax.numpy as jnp
from jax.experimental import pallas as pl


def kernel(x, c, W_qkv, b_qkv, W_proj, b_proj, W_ada, b_ada, W_gates, b_gates, W_c1, b_c1, W_c2, b_c2, W_gate_proj, W_up_proj, W_down_proj):
    raise NotImplementedError("write your pallas kernel here")


# speedup vs baseline: 1.7325x; 1.7325x over previous
import functools
import jax
import jax.numpy as jnp
from jax.experimental import pallas as pl
from jax.experimental.pallas import tpu as pltpu

B, S, D = 1, 2048, 1024
H, Dh, E = 16, 64, 8
I = 4096
K = (S // E) * 2
EPS = 1e-6
SM_SCALE = 1.0 / (Dh ** 0.5)
ST = 256
QT = 512
IT = 8
ITW = I // IT

F32 = jnp.float32


def _ln_mod(x, shift, scale):
    mu = jnp.mean(x, axis=-1, keepdims=True)
    var = jnp.mean((x - mu) ** 2, axis=-1, keepdims=True)
    xn = (x - mu) / jnp.sqrt(var + EPS)
    return xn * (1.0 + scale) + shift


def _ada_kernel(c_ref, W_ref, b_ref, out_ref):
    cs = jax.nn.silu(c_ref[...])
    out_ref[...] = jnp.dot(cs, W_ref[...], preferred_element_type=F32) + b_ref[...]


def _qkv_kernel(x_ref, ada_ref, W_ref, b_ref, out_ref):
    shift = ada_ref[0, 0:D][None]
    scale = ada_ref[0, D:2 * D][None]
    xm = _ln_mod(x_ref[...], shift, scale)
    out_ref[...] = jnp.dot(xm, W_ref[...], preferred_element_type=F32) + b_ref[...]


def _attn_kernel(q_ref, kv_ref, o_ref):
    cols = []
    for h in range(H):
        q = q_ref[:, h * Dh:(h + 1) * Dh]
        k = kv_ref[:, D + h * Dh:D + (h + 1) * Dh]
        v = kv_ref[:, 2 * D + h * Dh:2 * D + (h + 1) * Dh]
        s = jax.lax.dot_general(q, k, (((1,), (1,)), ((), ())),
                                preferred_element_type=F32) * SM_SCALE
        m = jnp.max(s, axis=-1, keepdims=True)
        p = jnp.exp(s - m)
        p = p / jnp.sum(p, axis=-1, keepdims=True)
        cols.append(jnp.dot(p, v, preferred_element_type=F32))
    o_ref[...] = jnp.concatenate(cols, axis=1)


def _post_kernel(x_ref, o_ref, Wp_ref, bp_ref, ada_ref, Wg_ref, bg_ref,
                 x1_ref, moe_ref, sc_ref):
    attn = jnp.dot(o_ref[...], Wp_ref[...], preferred_element_type=F32) + bp_ref[...]
    gate_msa = ada_ref[0, 2 * D:3 * D][None]
    x1 = x_ref[...] + gate_msa * attn
    x1_ref[...] = x1
    shift_mlp = ada_ref[0, 3 * D:4 * D][None]
    scale_mlp = ada_ref[0, 4 * D:5 * D][None]
    mi = _ln_mod(x1, shift_mlp, scale_mlp)
    moe_ref[...] = mi
    Wg = (Wg_ref[0] + Wg_ref[1] + Wg_ref[2] + Wg_ref[3]) * 0.25
    bg = ((bg_ref[0] + bg_ref[1] + bg_ref[2] + bg_ref[3]) * 0.25)[None]
    logits = jnp.dot(mi, Wg, preferred_element_type=F32) + bg
    lm = jnp.max(logits, axis=-1, keepdims=True)
    p = jnp.exp(logits - lm)
    sc_ref[...] = p / jnp.sum(p, axis=-1, keepdims=True)


def _cumsum_lanes(x):
    n = x.shape[-1]
    s = 1
    while s < n:
        shifted = jnp.concatenate(
            [jnp.zeros(x.shape[:-1] + (s,), x.dtype), x[..., :n - s]], axis=-1)
        x = x + shifted
        s *= 2
    return x


def _route_kernel(s_ref, mask_ref, rank_ref):
    s = s_ref[...]

    def body(_, carry):
        lo, hi = carry
        mid = (lo + hi) // 2
        t = jax.lax.bitcast_convert_type(mid, F32)
        cnt = jnp.sum((s > t).astype(jnp.int32), axis=-1, keepdims=True)
        pred = cnt >= K
        return jnp.where(pred, mid, lo), jnp.where(pred, hi, mid)

    lo0 = jnp.zeros((E, 1), jnp.int32)
    hi0 = jnp.full((E, 1), 0x3F800001, jnp.int32)
    lo, hi = jax.lax.fori_loop(0, 32, body, (lo0, hi0))
    T = jax.lax.bitcast_convert_type(hi, F32)
    gt = s > T
    cnt_g = jnp.sum(gt.astype(jnp.int32), axis=-1, keepdims=True)
    need = K - cnt_g
    eq = s == T
    eqi = eq.astype(jnp.int32)
    tie_rank = _cumsum_lanes(eqi) - eqi
    mask = gt | (eq & (tie_rank < need))
    mi = mask.astype(jnp.int32)
    rank = _cumsum_lanes(mi) - mi
    mask_ref[...] = mask.astype(F32)
    rank_ref[...] = rank


def _moe_kernel(mask_ref, rank_ref, sc_ref, mi_ref,
                Wg_ref, Wu_ref, Wd_ref, out_ref,
                P_ref, ei_ref, acc_ref, g_ref):
    e = pl.program_id(0)
    i = pl.program_id(1)

    @pl.when(jnp.logical_and(e == 0, i == 0))
    def _():
        out_ref[...] = jnp.zeros_like(out_ref)

    @pl.when(i == 0)
    def _():
        rank = rank_ref[0]
        m = mask_ref[0]
        jj = jax.lax.broadcasted_iota(jnp.int32, (K, S), 0)
        P = jnp.where((rank == jj) & (m > 0.0), 1.0, 0.0)
        P_ref[...] = P
        g_ref[...] = jnp.sum(P * sc_ref[0], axis=1, keepdims=True)
        ei_ref[...] = jnp.dot(P, mi_ref[...], preferred_element_type=F32)

    ei = ei_ref[...]
    hg = jnp.dot(ei, Wg_ref[0], preferred_element_type=F32)
    hu = jnp.dot(ei, Wu_ref[0], preferred_element_type=F32)
    h = jax.nn.silu(hg) * hu
    part = jnp.dot(h, Wd_ref[0], preferred_element_type=F32)

    @pl.when(i == 0)
    def _():
        acc_ref[...] = part

    @pl.when(i > 0)
    def _():
        acc_ref[...] += part

    @pl.when(i == IT - 1)
    def _():
        gated = g_ref[...] * acc_ref[...]
        out_ref[...] += jax.lax.dot_general(
            P_ref[...], gated, (((0,), (0,)), ((), ())),
            preferred_element_type=F32)


def _combine_kernel(x1_ref, y_ref, ada_ref, out_ref):
    gate_mlp = ada_ref[0, 5 * D:6 * D][None]
    out_ref[...] = x1_ref[...] + gate_mlp * y_ref[...]


def kernel(x, c, W_qkv, b_qkv, W_proj, b_proj, W_ada, b_ada, W_gates,
           b_gates, W_c1, b_c1, W_c2, b_c2, W_gate_proj, W_up_proj,
           W_down_proj):
    del W_c1, b_c1, W_c2, b_c2
    xf = x[0]

    ada = pl.pallas_call(
        _ada_kernel,
        grid=(6,),
        in_specs=[
            pl.BlockSpec((1, D), lambda j: (0, 0)),
            pl.BlockSpec((D, D), lambda j: (0, j)),
            pl.BlockSpec((1, D), lambda j: (0, j)),
        ],
        out_specs=pl.BlockSpec((1, D), lambda j: (0, j)),
        out_shape=jax.ShapeDtypeStruct((1, 6 * D), F32),
    )(c, W_ada, b_ada.reshape(1, -1))

    qkv = pl.pallas_call(
        _qkv_kernel,
        grid=(S // ST,),
        in_specs=[
            pl.BlockSpec((ST, D), lambda t: (t, 0)),
            pl.BlockSpec((1, 6 * D), lambda t: (0, 0)),
            pl.BlockSpec((D, 3 * D), lambda t: (0, 0)),
            pl.BlockSpec((1, 3 * D), lambda t: (0, 0)),
        ],
        out_specs=pl.BlockSpec((ST, 3 * D), lambda t: (t, 0)),
        out_shape=jax.ShapeDtypeStruct((S, 3 * D), F32),
    )(xf, ada, W_qkv, b_qkv.reshape(1, -1))

    o = pl.pallas_call(
        _attn_kernel,
        grid=(S // QT,),
        in_specs=[
            pl.BlockSpec((QT, 3 * D), lambda t: (t, 0)),
            pl.BlockSpec((S, 3 * D), lambda t: (0, 0)),
        ],
        out_specs=pl.BlockSpec((QT, D), lambda t: (t, 0)),
        out_shape=jax.ShapeDtypeStruct((S, D), F32),
    )(qkv, qkv)

    x1, moe_in, scores = pl.pallas_call(
        _post_kernel,
        grid=(S // ST,),
        in_specs=[
            pl.BlockSpec((ST, D), lambda t: (t, 0)),
            pl.BlockSpec((ST, D), lambda t: (t, 0)),
            pl.BlockSpec((D, D), lambda t: (0, 0)),
            pl.BlockSpec((1, D), lambda t: (0, 0)),
            pl.BlockSpec((1, 6 * D), lambda t: (0, 0)),
            pl.BlockSpec((4, D, E), lambda t: (0, 0, 0)),
            pl.BlockSpec((4, E), lambda t: (0, 0)),
        ],
        out_specs=[
            pl.BlockSpec((ST, D), lambda t: (t, 0)),
            pl.BlockSpec((ST, D), lambda t: (t, 0)),
            pl.BlockSpec((ST, E), lambda t: (t, 0)),
        ],
        out_shape=[
            jax.ShapeDtypeStruct((S, D), F32),
            jax.ShapeDtypeStruct((S, D), F32),
            jax.ShapeDtypeStruct((S, E), F32),
        ],
    )(xf, o, W_proj, b_proj.reshape(1, -1), ada, W_gates, b_gates)

    sT = scores.T

    mask, rank = pl.pallas_call(
        _route_kernel,
        in_specs=[pl.BlockSpec((E, S), lambda: (0, 0))],
        out_specs=[
            pl.BlockSpec((E, S), lambda: (0, 0)),
            pl.BlockSpec((E, S), lambda: (0, 0)),
        ],
        out_shape=[
            jax.ShapeDtypeStruct((E, S), F32),
            jax.ShapeDtypeStruct((E, S), jnp.int32),
        ],
    )(sT)

    y = pl.pallas_call(
        _moe_kernel,
        grid=(E, IT),
        in_specs=[
            pl.BlockSpec((1, 1, S), lambda e, i: (e, 0, 0)),
            pl.BlockSpec((1, 1, S), lambda e, i: (e, 0, 0)),
            pl.BlockSpec((1, 1, S), lambda e, i: (e, 0, 0)),
            pl.BlockSpec((S, D), lambda e, i: (0, 0)),
            pl.BlockSpec((1, D, ITW), lambda e, i: (e, 0, i)),
            pl.BlockSpec((1, D, ITW), lambda e, i: (e, 0, i)),
            pl.BlockSpec((1, ITW, D), lambda e, i: (e, i, 0)),
        ],
        out_specs=pl.BlockSpec((S, D), lambda e, i: (0, 0)),
        out_shape=jax.ShapeDtypeStruct((S, D), F32),
        scratch_shapes=[
            pltpu.VMEM((K, S), F32),
            pltpu.VMEM((K, D), F32),
            pltpu.VMEM((K, D), F32),
            pltpu.VMEM((K, 1), F32),
        ],
    )(mask.reshape(E, 1, S), rank.reshape(E, 1, S), sT.reshape(E, 1, S),
      moe_in, W_gate_proj, W_up_proj, W_down_proj)

    out = pl.pallas_call(
        _combine_kernel,
        grid=(S // ST,),
        in_specs=[
            pl.BlockSpec((ST, D), lambda t: (t, 0)),
            pl.BlockSpec((ST, D), lambda t: (t, 0)),
            pl.BlockSpec((1, 6 * D), lambda t: (0, 0)),
        ],
        out_specs=pl.BlockSpec((ST, D), lambda t: (t, 0)),
        out_shape=jax.ShapeDtypeStruct((S, D), F32),
    )(x1, y, ada)

    return out[None]
